# baseline (speedup 1.0000x reference)
import jax
import jax.numpy as jnp
from jax.experimental import pallas as pl
from jax.experimental.pallas import tpu as pltpu


def _mm_relu_body(msg_ref, emb_ref, w_ref, out_ref):
    x = msg_ref[0] + emb_ref[0]
    y = jax.lax.dot_general(
        x, w_ref[0],
        dimension_numbers=(((1,), (1,)), ((), ())),
        preferred_element_type=jnp.float32,
    )
    out_ref[0] = jnp.maximum(y, 0.0)


def _mm_relu(msg, emb, w):
    n = msg.shape[1]
    br = 2000
    grid = (2, n // br)
    return pl.pallas_call(
        _mm_relu_body,
        grid=grid,
        in_specs=[
            pl.BlockSpec((1, br, 128), lambda g, r: (g, r, 0)),
            pl.BlockSpec((1, br, 128), lambda g, r: (g, r, 0)),
            pl.BlockSpec((1, 128, 128), lambda g, r: (g, 0, 0)),
        ],
        out_specs=pl.BlockSpec((1, br, 128), lambda g, r: (g, r, 0)),
        out_shape=jax.ShapeDtypeStruct((2, n, 128), jnp.float32),
    )(msg, emb, w)


def kernel(u_emb, i_emb, edge_index, weights, W_u, W_i):
    user_idx = edge_index[0]
    item_idx = edge_index[1]
    num_users = u_emb.shape[0]
    num_items = i_emb.shape[0]
    user_deg = jnp.maximum(jnp.bincount(user_idx, length=num_users).astype(jnp.float32), 1.0)
    item_deg = jnp.maximum(jnp.bincount(item_idx, length=num_items).astype(jnp.float32), 1.0)
    norm = 1.0 / (jnp.sqrt(user_deg[user_idx]) * jnp.sqrt(item_deg[item_idx]))
    edge_weights = (weights * norm)[:, None]
    new_u_msg = jnp.zeros_like(u_emb).at[user_idx].add(i_emb[item_idx] * edge_weights)
    new_i_msg = jnp.zeros_like(i_emb).at[item_idx].add(u_emb[user_idx] * edge_weights)

    msg = jnp.stack([new_u_msg, new_i_msg])
    emb = jnp.stack([u_emb, i_emb])
    w = jnp.stack([W_u, W_i])
    out = _mm_relu(msg, emb, w)
    return (out[0], out[1])


# speedup vs baseline: 12.4345x; 12.4345x over previous
import functools

import jax
import jax.numpy as jnp
from jax import lax
from jax.experimental import pallas as pl
from jax.experimental.pallas import tpu as pltpu
from jax.experimental.pallas import tpu_sc as plsc

NC = 2
NS = 16
L = 16

N_NODES = 10000
N_PAD_NODES = 10240
EMB = 128
CHUNK = 128
G = 2048

_NO_LAYOUT = pltpu.CompilerParams(needs_layout_passes=False)


def _rsqrt_newton(d):
    xi = lax.bitcast_convert_type(d, jnp.int32)
    xi = 0x5F3759DF - lax.shift_right_logical(xi, 1)
    y = lax.bitcast_convert_type(xi, jnp.float32)
    for _ in range(3):
        y = y * (1.5 - 0.5 * d * y * y)
    return y


def _sc_factors(uidx, iidx, w, e_pad):
    ept16 = e_pad // NS
    mesh = plsc.VectorSubcoreMesh(core_axis_name="c", subcore_axis_name="s")

    @functools.partial(
        pl.kernel,
        out_type=jax.ShapeDtypeStruct((NC, e_pad // CHUNK, CHUNK), jnp.float32),
        mesh=mesh,
        compiler_params=_NO_LAYOUT,
        scratch_types=[
            pltpu.VMEM((G // CHUNK, CHUNK), jnp.int32),
            pltpu.VMEM((G // CHUNK, CHUNK), jnp.float32),
            pltpu.VMEM((G // CHUNK, CHUNK), jnp.float32),
            pltpu.VMEM((80, EMB), jnp.float32),
            pltpu.VMEM((80,), jnp.int32),
            pltpu.VMEM((8, EMB), jnp.float32),
            pltpu.VMEM_SHARED((80, EMB), jnp.float32),
        ],
    )
    def ka(idx_h, w_h, out_h, gidx, wloc, fout, hist, rowids, tbl, deg):
        c = lax.axis_index("c")
        s = lax.axis_index("s")
        pos16 = lax.iota(jnp.int32, L)
        zeros16 = jnp.zeros((L,), jnp.float32)

        def zero_body(r, _):
            for j in range(EMB // L):
                hist[r, pl.ds(j * L, L)] = zeros16
            return 0

        lax.fori_loop(0, 80, zero_body, 0)
        for r in range(5):
            rowids[pl.ds(r * L, L)] = pos16 + r * L
        for j in range(EMB // L):
            for r in range(5):
                tbl[r, pl.ds(j * L, L)] = zeros16
        pltpu.sync_copy(tbl.at[pl.ds(0, 5)], deg.at[pl.ds(s * 5, 5)])
        plsc.subcore_barrier()

        def deg_group(g_id, _):
            off = s * ept16 + g_id * G
            c0 = pl.multiple_of(off // CHUNK, 8)
            pltpu.sync_copy(idx_h.at[c, pl.ds(c0, G // CHUNK)], gidx)

            def chunk_body(kk, _):
                for g in range(CHUNK // L):
                    v = gidx[kk, pl.ds(g * L, L)]
                    cnt, last = plsc.scan_count(v)
                    plsc.addupdate_scatter(
                        hist,
                        [lax.shift_right_logical(v, 7), v & (EMB - 1)],
                        cnt.astype(jnp.float32), mask=last)
                return 0

            lax.fori_loop(0, G // CHUNK, chunk_body, 0)
            return 0

        lax.fori_loop(0, ept16 // G, deg_group, 0)
        pltpu.sync_copy(hist, deg.at[rowids], add=True)
        plsc.subcore_barrier()

        pltpu.sync_copy(deg.at[pl.ds(s * 5, 5)], tbl.at[pl.ds(0, 5)])

        def rsq_body(r, _):
            for j in range(EMB // L):
                d = jnp.maximum(tbl[r, pl.ds(j * L, L)], 1.0)
                tbl[r, pl.ds(j * L, L)] = _rsqrt_newton(d)
            return 0

        lax.fori_loop(0, 5, rsq_body, 0)
        pltpu.sync_copy(tbl.at[pl.ds(0, 5)], deg.at[pl.ds(s * 5, 5)])
        plsc.subcore_barrier()

        pltpu.sync_copy(deg, hist)

        def fac_group(g_id, _):
            off = s * ept16 + g_id * G
            c0 = pl.multiple_of(off // CHUNK, 8)
            pltpu.sync_copy(idx_h.at[c, pl.ds(c0, G // CHUNK)], gidx)
            pltpu.sync_copy(w_h.at[c, pl.ds(c0, G // CHUNK)], wloc)

            def chunk_body(kk, _):
                for g in range(CHUNK // L):
                    v = gidx[kk, pl.ds(g * L, L)]
                    f = plsc.load_gather(
                        hist,
                        [lax.shift_right_logical(v, 7), v & (EMB - 1)])
                    wv = wloc[kk, pl.ds(g * L, L)]
                    fout[kk, pl.ds(g * L, L)] = f * wv
                return 0

            lax.fori_loop(0, G // CHUNK, chunk_body, 0)
            pltpu.sync_copy(fout, out_h.at[c, pl.ds(c0, G // CHUNK)])
            return 0

        lax.fori_loop(0, ept16 // G, fac_group, 0)

    idx_stack = jnp.stack([uidx, iidx]).reshape(NC, e_pad // CHUNK, CHUNK)
    ones = jnp.ones_like(w)
    w_stack = jnp.stack([w, ones]).reshape(NC, e_pad // CHUNK, CHUNK)
    return ka(idx_stack, w_stack)


def _sc_messages(uidx, iidx, pwb, emb_stack, e_pad):
    ept = e_pad // NS
    ngrp = ept // G
    mesh = plsc.VectorSubcoreMesh(core_axis_name="c", subcore_axis_name="s")

    @functools.partial(
        pl.kernel,
        out_type=jax.ShapeDtypeStruct((NC, N_PAD_NODES, EMB), jnp.float32),
        mesh=mesh,
        compiler_params=_NO_LAYOUT,
        scratch_types=[
            pltpu.VMEM((G,), jnp.int32),
            pltpu.VMEM((G,), jnp.int32),
            pltpu.VMEM((G // CHUNK, CHUNK), jnp.float32),
            pltpu.VMEM((G // CHUNK, CHUNK), jnp.float32),
            pltpu.VMEM((CHUNK, EMB), jnp.float32),
            pltpu.VMEM((CHUNK,), jnp.float32),
            pltpu.VMEM((CHUNK,), jnp.int32),
            pltpu.VMEM((CHUNK,), jnp.int32),
            pltpu.VMEM_SHARED((N_PAD_NODES, EMB), jnp.float32),
            pltpu.SemaphoreType.DMA,
        ],
    )
    def kb(uidx_h, iidx_h, pwb_h, emb_h, out_h,
           uloc, iloc, pwl, pbl, rows, ew, gidx, didx, acc, sem):
        c = lax.axis_index("c")
        s = lax.axis_index("s")
        is_u = (c == 0).astype(jnp.int32)
        zeros16 = jnp.zeros((L,), jnp.float32)

        def zero_body(r, _):
            for j in range(EMB // L):
                rows[r, pl.ds(j * L, L)] = zeros16
            return 0

        lax.fori_loop(0, CHUNK, zero_body, 0)
        for m in range(5):
            pltpu.sync_copy(rows, acc.at[pl.ds((s * 5 + m) * CHUNK, CHUNK)])
        plsc.subcore_barrier()

        def msg_group(g_id, _):
            off = s * ept + g_id * G
            pltpu.sync_copy(uidx_h.at[pl.ds(off, G)], uloc)
            pltpu.sync_copy(iidx_h.at[pl.ds(off, G)], iloc)
            c0 = pl.multiple_of(off // CHUNK, 8)
            pltpu.sync_copy(pwb_h.at[0, pl.ds(c0, G // CHUNK)], pwl)
            pltpu.sync_copy(pwb_h.at[1, pl.ds(c0, G // CHUNK)], pbl)

            def chunk_body(kk, _):
                o = kk * CHUNK
                for g in range(CHUNK // L):
                    u = uloc[pl.ds(o + g * L, L)]
                    i = iloc[pl.ds(o + g * L, L)]
                    ew[pl.ds(g * L, L)] = (
                        pwl[kk, pl.ds(g * L, L)] * pbl[kk, pl.ds(g * L, L)])
                    gsel = u + (i - u) * is_u
                    dsel = i + (u - i) * is_u
                    gidx[pl.ds(g * L, L)] = gsel + N_PAD_NODES * is_u
                    didx[pl.ds(g * L, L)] = dsel
                pltpu.async_copy(emb_h.at[gidx], rows, sem).wait()

                def scale_body(g2, _):
                    evec = ew[pl.ds(g2 * L, L)]
                    for l in range(L):
                        sc = evec[l]
                        e = g2 * L + l
                        for j in range(EMB // L):
                            rows[e, pl.ds(j * L, L)] = (
                                rows[e, pl.ds(j * L, L)] * sc)
                    return 0

                lax.fori_loop(0, CHUNK // L, scale_body, 0)
                pltpu.sync_copy(rows, acc.at[didx], add=True)
                return 0

            lax.fori_loop(0, G // CHUNK, chunk_body, 0)
            return 0

        lax.fori_loop(0, ngrp, msg_group, 0)
        plsc.subcore_barrier()

        for m in range(5):
            r0 = (s * 5 + m) * CHUNK
            pltpu.sync_copy(acc.at[pl.ds(r0, CHUNK)], rows)
            pltpu.sync_copy(
                rows, out_h.at[c, pl.ds(pl.multiple_of(r0, 8), CHUNK)])

    return kb(uidx, iidx, pwb, emb_stack)


def _mm_relu_body(msg_ref, emb_ref, w_ref, out_ref):
    x = msg_ref[0] + emb_ref[0]
    y = lax.dot_general(
        x, w_ref[0],
        dimension_numbers=(((1,), (1,)), ((), ())),
        preferred_element_type=jnp.float32,
    )
    out_ref[0] = jnp.maximum(y, 0.0)


def _mm_relu(msg, emb, w):
    n = msg.shape[1]
    br = 2000
    return pl.pallas_call(
        _mm_relu_body,
        grid=(2, n // br),
        in_specs=[
            pl.BlockSpec((1, br, 128), lambda g, r: (g, r, 0)),
            pl.BlockSpec((1, br, 128), lambda g, r: (g, r, 0)),
            pl.BlockSpec((1, 128, 128), lambda g, r: (g, 0, 0)),
        ],
        out_specs=pl.BlockSpec((1, br, 128), lambda g, r: (g, r, 0)),
        out_shape=jax.ShapeDtypeStruct((2, n, 128), jnp.float32),
    )(msg, emb, w)


def kernel(u_emb, i_emb, edge_index, weights, W_u, W_i):
    e = edge_index.shape[1]
    per_round = G * NC * NS
    e_pad = -(-e // per_round) * per_round
    n_pad = e_pad - e

    uidx = edge_index[0]
    iidx = edge_index[1]
    if n_pad:
        pad_nodes = N_NODES + (
            jnp.arange(n_pad, dtype=jnp.int32) % (N_PAD_NODES - N_NODES))
        uidx = jnp.concatenate([uidx, pad_nodes])
        iidx = jnp.concatenate([iidx, pad_nodes])
        weights = jnp.concatenate([weights, jnp.zeros((n_pad,), jnp.float32)])

    pwb = _sc_factors(uidx, iidx, weights, e_pad)

    zpad = jnp.zeros((N_PAD_NODES - N_NODES, EMB), jnp.float32)
    emb_stack = jnp.concatenate([u_emb, zpad, i_emb, zpad])

    msg = _sc_messages(uidx, iidx, pwb, emb_stack, e_pad)
    msg = msg[:, :N_NODES]

    emb = jnp.stack([u_emb, i_emb])
    w = jnp.stack([W_u, W_i])
    out = _mm_relu(msg, emb, w)
    return (out[0], out[1])


# speedup vs baseline: 17.7928x; 1.4309x over previous
import functools

import jax
import jax.numpy as jnp
from jax import lax
from jax.experimental import pallas as pl
from jax.experimental.pallas import tpu as pltpu
from jax.experimental.pallas import tpu_sc as plsc

NC = 2
NS = 16
L = 16

N_NODES = 10000
N_PAD_NODES = 10240
EMB = 128
CHUNK = 128
G = 2048

_NO_LAYOUT = pltpu.CompilerParams(needs_layout_passes=False)


def _rsqrt_newton(d):
    xi = lax.bitcast_convert_type(d, jnp.int32)
    xi = 0x5F3759DF - lax.shift_right_logical(xi, 1)
    y = lax.bitcast_convert_type(xi, jnp.float32)
    for _ in range(3):
        y = y * (1.5 - 0.5 * d * y * y)
    return y


def _sc_factors(uidx, iidx, w, e_pad):
    ept16 = e_pad // NS
    mesh = plsc.VectorSubcoreMesh(core_axis_name="c", subcore_axis_name="s")

    @functools.partial(
        pl.kernel,
        out_type=jax.ShapeDtypeStruct((NC, e_pad // CHUNK, CHUNK), jnp.float32),
        mesh=mesh,
        compiler_params=_NO_LAYOUT,
        scratch_types=[
            pltpu.VMEM((G // CHUNK, CHUNK), jnp.int32),
            pltpu.VMEM((G // CHUNK, CHUNK), jnp.float32),
            pltpu.VMEM((G // CHUNK, CHUNK), jnp.float32),
            pltpu.VMEM((80, EMB), jnp.float32),
            pltpu.VMEM((80,), jnp.int32),
            pltpu.VMEM((8, EMB), jnp.float32),
            pltpu.VMEM_SHARED((80, EMB), jnp.float32),
        ],
    )
    def ka(idx_h, w_h, out_h, gidx, wloc, fout, hist, rowids, tbl, deg):
        c = lax.axis_index("c")
        s = lax.axis_index("s")
        pos16 = lax.iota(jnp.int32, L)
        zeros16 = jnp.zeros((L,), jnp.float32)

        def zero_body(r, _):
            for j in range(EMB // L):
                hist[r, pl.ds(j * L, L)] = zeros16
            return 0

        lax.fori_loop(0, 80, zero_body, 0)
        for r in range(5):
            rowids[pl.ds(r * L, L)] = pos16 + r * L
        for j in range(EMB // L):
            for r in range(5):
                tbl[r, pl.ds(j * L, L)] = zeros16
        pltpu.sync_copy(tbl.at[pl.ds(0, 5)], deg.at[pl.ds(s * 5, 5)])
        plsc.subcore_barrier()

        def deg_group(g_id, _):
            off = s * ept16 + g_id * G
            c0 = pl.multiple_of(off // CHUNK, 8)
            pltpu.sync_copy(idx_h.at[c, pl.ds(c0, G // CHUNK)], gidx)

            def chunk_body(kk, _):
                for g in range(CHUNK // L):
                    v = gidx[kk, pl.ds(g * L, L)]
                    cnt, last = plsc.scan_count(v)
                    plsc.addupdate_scatter(
                        hist,
                        [lax.shift_right_logical(v, 7), v & (EMB - 1)],
                        cnt.astype(jnp.float32), mask=last)
                return 0

            lax.fori_loop(0, G // CHUNK, chunk_body, 0)
            return 0

        lax.fori_loop(0, ept16 // G, deg_group, 0)
        pltpu.sync_copy(hist, deg.at[rowids], add=True)
        plsc.subcore_barrier()

        pltpu.sync_copy(deg.at[pl.ds(s * 5, 5)], tbl.at[pl.ds(0, 5)])

        def rsq_body(r, _):
            for j in range(EMB // L):
                d = jnp.maximum(tbl[r, pl.ds(j * L, L)], 1.0)
                tbl[r, pl.ds(j * L, L)] = _rsqrt_newton(d)
            return 0

        lax.fori_loop(0, 5, rsq_body, 0)
        pltpu.sync_copy(tbl.at[pl.ds(0, 5)], deg.at[pl.ds(s * 5, 5)])
        plsc.subcore_barrier()

        pltpu.sync_copy(deg, hist)

        def fac_group(g_id, _):
            off = s * ept16 + g_id * G
            c0 = pl.multiple_of(off // CHUNK, 8)
            pltpu.sync_copy(idx_h.at[c, pl.ds(c0, G // CHUNK)], gidx)
            pltpu.sync_copy(w_h.at[c, pl.ds(c0, G // CHUNK)], wloc)

            def chunk_body(kk, _):
                for g in range(CHUNK // L):
                    v = gidx[kk, pl.ds(g * L, L)]
                    f = plsc.load_gather(
                        hist,
                        [lax.shift_right_logical(v, 7), v & (EMB - 1)])
                    wv = wloc[kk, pl.ds(g * L, L)]
                    fout[kk, pl.ds(g * L, L)] = f * wv
                return 0

            lax.fori_loop(0, G // CHUNK, chunk_body, 0)
            pltpu.sync_copy(fout, out_h.at[c, pl.ds(c0, G // CHUNK)])
            return 0

        lax.fori_loop(0, ept16 // G, fac_group, 0)

    idx_stack = jnp.stack([uidx, iidx]).reshape(NC, e_pad // CHUNK, CHUNK)
    ones = jnp.ones_like(w)
    w_stack = jnp.stack([w, ones]).reshape(NC, e_pad // CHUNK, CHUNK)
    return ka(idx_stack, w_stack)


def _sc_messages(uidx, iidx, pwb, emb_stack, e_pad):
    ept = e_pad // NS
    ngrp = ept // G
    npair = G // CHUNK // 2
    mesh = plsc.VectorSubcoreMesh(core_axis_name="c", subcore_axis_name="s")

    @functools.partial(
        pl.kernel,
        out_type=jax.ShapeDtypeStruct((NC, N_PAD_NODES, EMB), jnp.float32),
        mesh=mesh,
        compiler_params=_NO_LAYOUT,
        scratch_types=[
            pltpu.VMEM((G,), jnp.int32),
            pltpu.VMEM((G,), jnp.int32),
            pltpu.VMEM((G // CHUNK, CHUNK), jnp.float32),
            pltpu.VMEM((G // CHUNK, CHUNK), jnp.float32),
            pltpu.VMEM((CHUNK, EMB), jnp.float32),
            pltpu.VMEM((CHUNK, EMB), jnp.float32),
            pltpu.VMEM((CHUNK,), jnp.float32),
            pltpu.VMEM((CHUNK,), jnp.float32),
            pltpu.VMEM((CHUNK,), jnp.int32),
            pltpu.VMEM((CHUNK,), jnp.int32),
            pltpu.VMEM((CHUNK,), jnp.int32),
            pltpu.VMEM((CHUNK,), jnp.int32),
            pltpu.VMEM_SHARED((N_PAD_NODES, EMB), jnp.float32),
            pltpu.SemaphoreType.DMA,
            pltpu.SemaphoreType.DMA,
        ],
    )
    def kb(uidx_h, iidx_h, pwb_h, emb_h, out_h,
           uloc, iloc, pwl, pbl, rows0, rows1, ew0, ew1, gidx0, gidx1,
           didx0, didx1, acc, sem0, sem1):
        c = lax.axis_index("c")
        s = lax.axis_index("s")
        is_u = (c == 0).astype(jnp.int32)
        zeros16 = jnp.zeros((L,), jnp.float32)
        bufs = ((rows0, ew0, gidx0, didx0, sem0),
                (rows1, ew1, gidx1, didx1, sem1))

        def prep(kk, b):
            rows, ew, gidx, didx, sem = bufs[b]
            for g in range(CHUNK // L):
                u = uloc[pl.ds(kk * CHUNK + g * L, L)]
                i = iloc[pl.ds(kk * CHUNK + g * L, L)]
                ew[pl.ds(g * L, L)] = (
                    pwl[kk, pl.ds(g * L, L)] * pbl[kk, pl.ds(g * L, L)])
                gsel = u + (i - u) * is_u
                dsel = i + (u - i) * is_u
                gidx[pl.ds(g * L, L)] = gsel + N_PAD_NODES * is_u
                didx[pl.ds(g * L, L)] = dsel
            pltpu.async_copy(emb_h.at[gidx], rows, sem)

        def drain(b):
            rows, ew, gidx, didx, sem = bufs[b]
            pltpu.make_async_copy(emb_h.at[gidx], rows, sem).wait()

        def scale_scatter(b):
            rows, ew, gidx, didx, sem = bufs[b]

            def scale_body(g2, _):
                evec = ew[pl.ds(g2 * L, L)]
                for l in range(L):
                    sc = evec[l]
                    e = g2 * L + l
                    for j in range(EMB // L):
                        rows[e, pl.ds(j * L, L)] = (
                            rows[e, pl.ds(j * L, L)] * sc)
                return 0

            lax.fori_loop(0, CHUNK // L, scale_body, 0)
            pltpu.sync_copy(rows, acc.at[didx], add=True)

        def zero_body(r, _):
            for j in range(EMB // L):
                rows0[r, pl.ds(j * L, L)] = zeros16
            return 0

        lax.fori_loop(0, CHUNK, zero_body, 0)
        for m in range(5):
            pltpu.sync_copy(rows0, acc.at[pl.ds((s * 5 + m) * CHUNK, CHUNK)])
        plsc.subcore_barrier()

        def msg_group(g_id, _):
            off = s * ept + g_id * G
            pltpu.sync_copy(uidx_h.at[pl.ds(off, G)], uloc)
            pltpu.sync_copy(iidx_h.at[pl.ds(off, G)], iloc)
            c0 = pl.multiple_of(off // CHUNK, 8)
            pltpu.sync_copy(pwb_h.at[0, pl.ds(c0, G // CHUNK)], pwl)
            pltpu.sync_copy(pwb_h.at[1, pl.ds(c0, G // CHUNK)], pbl)
            prep(0, 0)

            def pair_body(k2, _):
                drain(0)
                prep(2 * k2 + 1, 1)
                scale_scatter(0)
                drain(1)

                @pl.when(k2 < npair - 1)
                def _():
                    prep(2 * k2 + 2, 0)

                scale_scatter(1)
                return 0

            lax.fori_loop(0, npair, pair_body, 0)
            return 0

        lax.fori_loop(0, ngrp, msg_group, 0)
        plsc.subcore_barrier()

        for m in range(5):
            r0 = (s * 5 + m) * CHUNK
            pltpu.sync_copy(
                acc.at[pl.ds(r0, CHUNK)],
                out_h.at[c, pl.ds(pl.multiple_of(r0, 8), CHUNK)])

    return kb(uidx, iidx, pwb, emb_stack)


def _mm_relu_body(msg_ref, emb_ref, w_ref, out_ref):
    x = msg_ref[0] + emb_ref[0]
    y = lax.dot_general(
        x, w_ref[0],
        dimension_numbers=(((1,), (1,)), ((), ())),
        preferred_element_type=jnp.float32,
    )
    out_ref[0] = jnp.maximum(y, 0.0)


def _mm_relu(msg, emb, w):
    n = msg.shape[1]
    br = 2000
    return pl.pallas_call(
        _mm_relu_body,
        grid=(2, n // br),
        in_specs=[
            pl.BlockSpec((1, br, 128), lambda g, r: (g, r, 0)),
            pl.BlockSpec((1, br, 128), lambda g, r: (g, r, 0)),
            pl.BlockSpec((1, 128, 128), lambda g, r: (g, 0, 0)),
        ],
        out_specs=pl.BlockSpec((1, br, 128), lambda g, r: (g, r, 0)),
        out_shape=jax.ShapeDtypeStruct((2, n, 128), jnp.float32),
    )(msg, emb, w)


def kernel(u_emb, i_emb, edge_index, weights, W_u, W_i):
    e = edge_index.shape[1]
    per_round = G * NC * NS
    e_pad = -(-e // per_round) * per_round
    n_pad = e_pad - e

    uidx = edge_index[0]
    iidx = edge_index[1]
    if n_pad:
        pad_nodes = N_NODES + (
            jnp.arange(n_pad, dtype=jnp.int32) % (N_PAD_NODES - N_NODES))
        uidx = jnp.concatenate([uidx, pad_nodes])
        iidx = jnp.concatenate([iidx, pad_nodes])
        weights = jnp.concatenate([weights, jnp.zeros((n_pad,), jnp.float32)])

    pwb = _sc_factors(uidx, iidx, weights, e_pad)

    zpad = jnp.zeros((N_PAD_NODES - N_NODES, EMB), jnp.float32)
    emb_stack = jnp.concatenate([u_emb, zpad, i_emb, zpad])

    msg = _sc_messages(uidx, iidx, pwb, emb_stack, e_pad)
    msg = msg[:, :N_NODES]

    emb = jnp.stack([u_emb, i_emb])
    w = jnp.stack([W_u, W_i])
    out = _mm_relu(msg, emb, w)
    return (out[0], out[1])


# speedup vs baseline: 17.8839x; 1.0051x over previous
import functools

import jax
import jax.numpy as jnp
from jax import lax
from jax.experimental import pallas as pl
from jax.experimental.pallas import tpu as pltpu
from jax.experimental.pallas import tpu_sc as plsc

NC = 2
NS = 16
L = 16

N_NODES = 10000
N_PAD_NODES = 10240
EMB = 128
CHUNK = 128
G = 2048

_NO_LAYOUT = pltpu.CompilerParams(needs_layout_passes=False)


def _rsqrt_newton(d):
    xi = lax.bitcast_convert_type(d, jnp.int32)
    xi = 0x5F3759DF - lax.shift_right_logical(xi, 1)
    y = lax.bitcast_convert_type(xi, jnp.float32)
    for _ in range(3):
        y = y * (1.5 - 0.5 * d * y * y)
    return y


def _sc_factors(uidx, iidx, w, e_pad):
    ept16 = e_pad // NS
    mesh = plsc.VectorSubcoreMesh(core_axis_name="c", subcore_axis_name="s")

    @functools.partial(
        pl.kernel,
        out_type=jax.ShapeDtypeStruct((NC, e_pad // CHUNK, CHUNK), jnp.float32),
        mesh=mesh,
        compiler_params=_NO_LAYOUT,
        scratch_types=[
            pltpu.VMEM((G // CHUNK, CHUNK), jnp.int32),
            pltpu.VMEM((G // CHUNK, CHUNK), jnp.float32),
            pltpu.VMEM((G // CHUNK, CHUNK), jnp.float32),
            pltpu.VMEM((80, EMB), jnp.float32),
            pltpu.VMEM((80,), jnp.int32),
            pltpu.VMEM((8, EMB), jnp.float32),
            pltpu.VMEM_SHARED((80, EMB), jnp.float32),
        ],
    )
    def ka(idx_h, w_h, out_h, gidx, wloc, fout, hist, rowids, tbl, deg):
        c = lax.axis_index("c")
        s = lax.axis_index("s")
        pos16 = lax.iota(jnp.int32, L)
        zeros16 = jnp.zeros((L,), jnp.float32)

        def zero_body(r, _):
            for j in range(EMB // L):
                hist[r, pl.ds(j * L, L)] = zeros16
            return 0

        lax.fori_loop(0, 80, zero_body, 0)
        for r in range(5):
            rowids[pl.ds(r * L, L)] = pos16 + r * L
        for j in range(EMB // L):
            for r in range(5):
                tbl[r, pl.ds(j * L, L)] = zeros16
        pltpu.sync_copy(tbl.at[pl.ds(0, 5)], deg.at[pl.ds(s * 5, 5)])
        plsc.subcore_barrier()

        def deg_group(g_id, _):
            off = s * ept16 + g_id * G
            c0 = pl.multiple_of(off // CHUNK, 8)
            pltpu.sync_copy(idx_h.at[c, pl.ds(c0, G // CHUNK)], gidx)

            def chunk_body(kk, _):
                for g in range(CHUNK // L):
                    v = gidx[kk, pl.ds(g * L, L)]
                    cnt, last = plsc.scan_count(v)
                    plsc.addupdate_scatter(
                        hist,
                        [lax.shift_right_logical(v, 7), v & (EMB - 1)],
                        cnt.astype(jnp.float32), mask=last)
                return 0

            lax.fori_loop(0, G // CHUNK, chunk_body, 0)
            return 0

        lax.fori_loop(0, ept16 // G, deg_group, 0)
        pltpu.sync_copy(hist, deg.at[rowids], add=True)
        plsc.subcore_barrier()

        pltpu.sync_copy(deg.at[pl.ds(s * 5, 5)], tbl.at[pl.ds(0, 5)])

        def rsq_body(r, _):
            for j in range(EMB // L):
                d = jnp.maximum(tbl[r, pl.ds(j * L, L)], 1.0)
                tbl[r, pl.ds(j * L, L)] = _rsqrt_newton(d)
            return 0

        lax.fori_loop(0, 5, rsq_body, 0)
        pltpu.sync_copy(tbl.at[pl.ds(0, 5)], deg.at[pl.ds(s * 5, 5)])
        plsc.subcore_barrier()

        pltpu.sync_copy(deg, hist)

        def fac_group(g_id, _):
            off = s * ept16 + g_id * G
            c0 = pl.multiple_of(off // CHUNK, 8)
            pltpu.sync_copy(idx_h.at[c, pl.ds(c0, G // CHUNK)], gidx)
            pltpu.sync_copy(w_h.at[c, pl.ds(c0, G // CHUNK)], wloc)

            def chunk_body(kk, _):
                for g in range(CHUNK // L):
                    v = gidx[kk, pl.ds(g * L, L)]
                    f = plsc.load_gather(
                        hist,
                        [lax.shift_right_logical(v, 7), v & (EMB - 1)])
                    wv = wloc[kk, pl.ds(g * L, L)]
                    fout[kk, pl.ds(g * L, L)] = f * wv
                return 0

            lax.fori_loop(0, G // CHUNK, chunk_body, 0)
            pltpu.sync_copy(fout, out_h.at[c, pl.ds(c0, G // CHUNK)])
            return 0

        lax.fori_loop(0, ept16 // G, fac_group, 0)

    idx_stack = jnp.stack([uidx, iidx]).reshape(NC, e_pad // CHUNK, CHUNK)
    ones = jnp.ones_like(w)
    w_stack = jnp.stack([w, ones]).reshape(NC, e_pad // CHUNK, CHUNK)
    return ka(idx_stack, w_stack)


def _sc_messages(uidx, iidx, pwb, emb_stack, e_pad):
    ept = e_pad // NS
    ngrp = ept // G
    npair = G // CHUNK // 2
    mesh = plsc.VectorSubcoreMesh(core_axis_name="c", subcore_axis_name="s")

    @functools.partial(
        pl.kernel,
        out_type=jax.ShapeDtypeStruct((NC, N_PAD_NODES, EMB), jnp.float32),
        mesh=mesh,
        compiler_params=_NO_LAYOUT,
        scratch_types=[
            pltpu.VMEM((G,), jnp.int32),
            pltpu.VMEM((G,), jnp.int32),
            pltpu.VMEM((G // CHUNK, CHUNK), jnp.float32),
            pltpu.VMEM((G // CHUNK, CHUNK), jnp.float32),
            pltpu.VMEM((CHUNK, EMB), jnp.float32),
            pltpu.VMEM((CHUNK, EMB), jnp.float32),
            pltpu.VMEM((CHUNK,), jnp.float32),
            pltpu.VMEM((CHUNK,), jnp.float32),
            pltpu.VMEM((CHUNK,), jnp.int32),
            pltpu.VMEM((CHUNK,), jnp.int32),
            pltpu.VMEM((CHUNK,), jnp.int32),
            pltpu.VMEM((CHUNK,), jnp.int32),
            pltpu.VMEM_SHARED((N_PAD_NODES, EMB), jnp.float32),
            pltpu.SemaphoreType.DMA,
            pltpu.SemaphoreType.DMA,
            pltpu.SemaphoreType.DMA,
            pltpu.SemaphoreType.DMA,
        ],
    )
    def kb(uidx_h, iidx_h, pwb_h, emb_h, out_h,
           uloc, iloc, pwl, pbl, rows0, rows1, ew0, ew1, gidx0, gidx1,
           didx0, didx1, acc, sem0, sem1, ssem0, ssem1):
        c = lax.axis_index("c")
        s = lax.axis_index("s")
        is_u = (c == 0).astype(jnp.int32)
        zeros16 = jnp.zeros((L,), jnp.float32)
        bufs = ((rows0, ew0, gidx0, didx0, sem0, ssem0),
                (rows1, ew1, gidx1, didx1, sem1, ssem1))

        def prep(kk, b):
            rows, ew, gidx, didx, sem, ssem = bufs[b]
            for g in range(CHUNK // L):
                u = uloc[pl.ds(kk * CHUNK + g * L, L)]
                i = iloc[pl.ds(kk * CHUNK + g * L, L)]
                ew[pl.ds(g * L, L)] = (
                    pwl[kk, pl.ds(g * L, L)] * pbl[kk, pl.ds(g * L, L)])
                gsel = u + (i - u) * is_u
                dsel = i + (u - i) * is_u
                gidx[pl.ds(g * L, L)] = gsel + N_PAD_NODES * is_u
                didx[pl.ds(g * L, L)] = dsel
            pltpu.async_copy(emb_h.at[gidx], rows, sem)

        def drain(b):
            rows, ew, gidx, didx, sem, ssem = bufs[b]
            pltpu.make_async_copy(emb_h.at[gidx], rows, sem).wait()

        def wait_scatter(b):
            rows, ew, gidx, didx, sem, ssem = bufs[b]
            pltpu.make_async_copy(rows, acc.at[didx], ssem).wait()

        def scale_scatter(b):
            rows, ew, gidx, didx, sem, ssem = bufs[b]

            def scale_body(g2, _):
                evec = ew[pl.ds(g2 * L, L)]
                for l in range(L):
                    sc = evec[l]
                    e = g2 * L + l
                    for j in range(EMB // L):
                        rows[e, pl.ds(j * L, L)] = (
                            rows[e, pl.ds(j * L, L)] * sc)
                return 0

            lax.fori_loop(0, CHUNK // L, scale_body, 0)
            pltpu.async_copy(rows, acc.at[didx], ssem, add=True)

        def zero_body(r, _):
            for j in range(EMB // L):
                rows0[r, pl.ds(j * L, L)] = zeros16
                rows1[r, pl.ds(j * L, L)] = zeros16
            return 0

        lax.fori_loop(0, CHUNK, zero_body, 0)
        for g in range(CHUNK // L):
            didx1[pl.ds(g * L, L)] = jnp.zeros((L,), jnp.int32)
        for m in range(5):
            pltpu.sync_copy(rows0, acc.at[pl.ds((s * 5 + m) * CHUNK, CHUNK)])
        plsc.subcore_barrier()
        pltpu.async_copy(rows1, acc.at[didx1], ssem1, add=True)

        def msg_group(g_id, _):
            off = s * ept + g_id * G
            pltpu.sync_copy(uidx_h.at[pl.ds(off, G)], uloc)
            pltpu.sync_copy(iidx_h.at[pl.ds(off, G)], iloc)
            c0 = pl.multiple_of(off // CHUNK, 8)
            pltpu.sync_copy(pwb_h.at[0, pl.ds(c0, G // CHUNK)], pwl)
            pltpu.sync_copy(pwb_h.at[1, pl.ds(c0, G // CHUNK)], pbl)
            prep(0, 0)

            def pair_body(k2, _):
                drain(0)
                wait_scatter(1)
                prep(2 * k2 + 1, 1)
                scale_scatter(0)
                drain(1)
                wait_scatter(0)

                @pl.when(k2 < npair - 1)
                def _():
                    prep(2 * k2 + 2, 0)

                scale_scatter(1)
                return 0

            lax.fori_loop(0, npair, pair_body, 0)
            return 0

        lax.fori_loop(0, ngrp, msg_group, 0)
        wait_scatter(1)
        plsc.subcore_barrier()

        for m in range(5):
            r0 = (s * 5 + m) * CHUNK
            pltpu.sync_copy(
                acc.at[pl.ds(r0, CHUNK)],
                out_h.at[c, pl.ds(pl.multiple_of(r0, 8), CHUNK)])

    return kb(uidx, iidx, pwb, emb_stack)


def _mm_relu_body(msg_ref, emb_ref, w_ref, out_ref):
    x = msg_ref[0] + emb_ref[0]
    y = lax.dot_general(
        x, w_ref[0],
        dimension_numbers=(((1,), (1,)), ((), ())),
        preferred_element_type=jnp.float32,
    )
    out_ref[0] = jnp.maximum(y, 0.0)


def _mm_relu(msg, emb, w):
    n = msg.shape[1]
    br = 2000
    return pl.pallas_call(
        _mm_relu_body,
        grid=(2, n // br),
        in_specs=[
            pl.BlockSpec((1, br, 128), lambda g, r: (g, r, 0)),
            pl.BlockSpec((1, br, 128), lambda g, r: (g, r, 0)),
            pl.BlockSpec((1, 128, 128), lambda g, r: (g, 0, 0)),
        ],
        out_specs=pl.BlockSpec((1, br, 128), lambda g, r: (g, r, 0)),
        out_shape=jax.ShapeDtypeStruct((2, n, 128), jnp.float32),
    )(msg, emb, w)


def kernel(u_emb, i_emb, edge_index, weights, W_u, W_i):
    e = edge_index.shape[1]
    per_round = G * NC * NS
    e_pad = -(-e // per_round) * per_round
    n_pad = e_pad - e

    uidx = edge_index[0]
    iidx = edge_index[1]
    if n_pad:
        pad_nodes = N_NODES + (
            jnp.arange(n_pad, dtype=jnp.int32) % (N_PAD_NODES - N_NODES))
        uidx = jnp.concatenate([uidx, pad_nodes])
        iidx = jnp.concatenate([iidx, pad_nodes])
        weights = jnp.concatenate([weights, jnp.zeros((n_pad,), jnp.float32)])

    pwb = _sc_factors(uidx, iidx, weights, e_pad)

    zpad = jnp.zeros((N_PAD_NODES - N_NODES, EMB), jnp.float32)
    emb_stack = jnp.concatenate([u_emb, zpad, i_emb, zpad])

    msg = _sc_messages(uidx, iidx, pwb, emb_stack, e_pad)
    msg = msg[:, :N_NODES]

    emb = jnp.stack([u_emb, i_emb])
    w = jnp.stack([W_u, W_i])
    out = _mm_relu(msg, emb, w)
    return (out[0], out[1])


# speedup vs baseline: 18.7424x; 1.0480x over previous
import functools

import jax
import jax.numpy as jnp
from jax import lax
from jax.experimental import pallas as pl
from jax.experimental.pallas import tpu as pltpu
from jax.experimental.pallas import tpu_sc as plsc

NC = 2
NS = 16
L = 16

N_NODES = 10000
N_PAD_NODES = 10240
EMB = 128
CHUNK = 128
G = 2048

_NO_LAYOUT = pltpu.CompilerParams(needs_layout_passes=False)


def _rsqrt_newton(d):
    xi = lax.bitcast_convert_type(d, jnp.int32)
    xi = 0x5F3759DF - lax.shift_right_logical(xi, 1)
    y = lax.bitcast_convert_type(xi, jnp.float32)
    for _ in range(3):
        y = y * (1.5 - 0.5 * d * y * y)
    return y


def _sc_factors(uidx, iidx, w, e_pad):
    ept16 = e_pad // NS
    mesh = plsc.VectorSubcoreMesh(core_axis_name="c", subcore_axis_name="s")

    @functools.partial(
        pl.kernel,
        out_type=jax.ShapeDtypeStruct((NC, e_pad // CHUNK, CHUNK), jnp.float32),
        mesh=mesh,
        compiler_params=_NO_LAYOUT,
        scratch_types=[
            pltpu.VMEM((ept16 // CHUNK, CHUNK), jnp.int32),
            pltpu.VMEM((ept16 // CHUNK, CHUNK), jnp.float32),
            pltpu.VMEM((ept16 // CHUNK, CHUNK), jnp.float32),
            pltpu.VMEM((80, EMB), jnp.float32),
            pltpu.VMEM((80,), jnp.int32),
            pltpu.VMEM((8, EMB), jnp.float32),
            pltpu.VMEM_SHARED((80, EMB), jnp.float32),
        ],
    )
    def ka(idx_h, w_h, out_h, gidx, wloc, fout, hist, rowids, tbl, deg):
        c = lax.axis_index("c")
        s = lax.axis_index("s")
        pos16 = lax.iota(jnp.int32, L)
        zeros16 = jnp.zeros((L,), jnp.float32)
        nck = ept16 // CHUNK
        c0 = pl.multiple_of((s * ept16) // CHUNK, 8)

        pltpu.sync_copy(idx_h.at[c, pl.ds(c0, nck)], gidx)
        pltpu.sync_copy(w_h.at[c, pl.ds(c0, nck)], wloc)

        def zero_body(r, _):
            for j in range(EMB // L):
                hist[r, pl.ds(j * L, L)] = zeros16
            return 0

        lax.fori_loop(0, 80, zero_body, 0)
        for r in range(5):
            rowids[pl.ds(r * L, L)] = pos16 + r * L
        for j in range(EMB // L):
            for r in range(5):
                tbl[r, pl.ds(j * L, L)] = zeros16
        pltpu.sync_copy(tbl.at[pl.ds(0, 5)], deg.at[pl.ds(s * 5, 5)])
        plsc.subcore_barrier()

        def chunk_body(kk, _):
            for g in range(CHUNK // L):
                v = gidx[kk, pl.ds(g * L, L)]
                cnt, last = plsc.scan_count(v)
                plsc.addupdate_scatter(
                    hist,
                    [lax.shift_right_logical(v, 7), v & (EMB - 1)],
                    cnt.astype(jnp.float32), mask=last)
            return 0

        lax.fori_loop(0, nck, chunk_body, 0)
        pltpu.sync_copy(hist, deg.at[rowids], add=True)
        plsc.subcore_barrier()

        pltpu.sync_copy(deg.at[pl.ds(s * 5, 5)], tbl.at[pl.ds(0, 5)])

        def rsq_body(r, _):
            for j in range(EMB // L):
                d = jnp.maximum(tbl[r, pl.ds(j * L, L)], 1.0)
                tbl[r, pl.ds(j * L, L)] = _rsqrt_newton(d)
            return 0

        lax.fori_loop(0, 5, rsq_body, 0)
        pltpu.sync_copy(tbl.at[pl.ds(0, 5)], deg.at[pl.ds(s * 5, 5)])
        plsc.subcore_barrier()

        pltpu.sync_copy(deg, hist)

        def fac_body(kk, _):
            for g in range(CHUNK // L):
                v = gidx[kk, pl.ds(g * L, L)]
                f = plsc.load_gather(
                    hist,
                    [lax.shift_right_logical(v, 7), v & (EMB - 1)])
                wv = wloc[kk, pl.ds(g * L, L)]
                fout[kk, pl.ds(g * L, L)] = f * wv
            return 0

        lax.fori_loop(0, nck, fac_body, 0)
        pltpu.sync_copy(fout, out_h.at[c, pl.ds(c0, nck)])

    idx_stack = jnp.stack([uidx, iidx]).reshape(NC, e_pad // CHUNK, CHUNK)
    ones = jnp.ones_like(w)
    w_stack = jnp.stack([w, ones]).reshape(NC, e_pad // CHUNK, CHUNK)
    return ka(idx_stack, w_stack)


def _sc_messages(uidx, iidx, pwb, emb_stack, e_pad):
    ept = e_pad // NS
    ngrp = ept // G
    npair = G // CHUNK // 2
    mesh = plsc.VectorSubcoreMesh(core_axis_name="c", subcore_axis_name="s")

    @functools.partial(
        pl.kernel,
        out_type=jax.ShapeDtypeStruct((NC, N_PAD_NODES, EMB), jnp.float32),
        mesh=mesh,
        compiler_params=_NO_LAYOUT,
        scratch_types=[
            pltpu.VMEM((G,), jnp.int32),
            pltpu.VMEM((G,), jnp.int32),
            pltpu.VMEM((G // CHUNK, CHUNK), jnp.float32),
            pltpu.VMEM((G // CHUNK, CHUNK), jnp.float32),
            pltpu.VMEM((CHUNK, EMB), jnp.float32),
            pltpu.VMEM((CHUNK, EMB), jnp.float32),
            pltpu.VMEM((CHUNK,), jnp.float32),
            pltpu.VMEM((CHUNK,), jnp.float32),
            pltpu.VMEM((CHUNK,), jnp.int32),
            pltpu.VMEM((CHUNK,), jnp.int32),
            pltpu.VMEM((CHUNK,), jnp.int32),
            pltpu.VMEM((CHUNK,), jnp.int32),
            pltpu.VMEM_SHARED((N_PAD_NODES, EMB), jnp.float32),
            pltpu.SemaphoreType.DMA,
            pltpu.SemaphoreType.DMA,
            pltpu.SemaphoreType.DMA,
            pltpu.SemaphoreType.DMA,
        ],
    )
    def kb(uidx_h, iidx_h, pwb_h, emb_h, out_h,
           uloc, iloc, pwl, pbl, rows0, rows1, ew0, ew1, gidx0, gidx1,
           didx0, didx1, acc, sem0, sem1, ssem0, ssem1):
        c = lax.axis_index("c")
        s = lax.axis_index("s")
        is_u = (c == 0).astype(jnp.int32)
        zeros16 = jnp.zeros((L,), jnp.float32)
        bufs = ((rows0, ew0, gidx0, didx0, sem0, ssem0),
                (rows1, ew1, gidx1, didx1, sem1, ssem1))

        def prep(kk, b):
            rows, ew, gidx, didx, sem, ssem = bufs[b]
            for g in range(CHUNK // L):
                u = uloc[pl.ds(kk * CHUNK + g * L, L)]
                i = iloc[pl.ds(kk * CHUNK + g * L, L)]
                ew[pl.ds(g * L, L)] = (
                    pwl[kk, pl.ds(g * L, L)] * pbl[kk, pl.ds(g * L, L)])
                gsel = u + (i - u) * is_u
                dsel = i + (u - i) * is_u
                gidx[pl.ds(g * L, L)] = gsel + N_PAD_NODES * is_u
                didx[pl.ds(g * L, L)] = dsel
            pltpu.async_copy(emb_h.at[gidx], rows, sem)

        def drain(b):
            rows, ew, gidx, didx, sem, ssem = bufs[b]
            pltpu.make_async_copy(emb_h.at[gidx], rows, sem).wait()

        def wait_scatter(b):
            rows, ew, gidx, didx, sem, ssem = bufs[b]
            pltpu.make_async_copy(rows, acc.at[didx], ssem).wait()

        def scale_scatter(b):
            rows, ew, gidx, didx, sem, ssem = bufs[b]

            def scale_body(g2, _):
                evec = ew[pl.ds(g2 * L, L)]
                for l in range(L):
                    sc = evec[l]
                    e = g2 * L + l
                    for j in range(EMB // L):
                        rows[e, pl.ds(j * L, L)] = (
                            rows[e, pl.ds(j * L, L)] * sc)
                return 0

            lax.fori_loop(0, CHUNK // L, scale_body, 0)
            pltpu.async_copy(rows, acc.at[didx], ssem, add=True)

        def zero_body(r, _):
            for j in range(EMB // L):
                rows0[r, pl.ds(j * L, L)] = zeros16
                rows1[r, pl.ds(j * L, L)] = zeros16
            return 0

        lax.fori_loop(0, CHUNK, zero_body, 0)
        for g in range(CHUNK // L):
            didx1[pl.ds(g * L, L)] = jnp.zeros((L,), jnp.int32)
        for m in range(5):
            pltpu.sync_copy(rows0, acc.at[pl.ds((s * 5 + m) * CHUNK, CHUNK)])
        plsc.subcore_barrier()
        pltpu.async_copy(rows1, acc.at[didx1], ssem1, add=True)

        def msg_group(g_id, _):
            off = s * ept + g_id * G
            pltpu.sync_copy(uidx_h.at[pl.ds(off, G)], uloc)
            pltpu.sync_copy(iidx_h.at[pl.ds(off, G)], iloc)
            c0 = pl.multiple_of(off // CHUNK, 8)
            pltpu.sync_copy(pwb_h.at[0, pl.ds(c0, G // CHUNK)], pwl)
            pltpu.sync_copy(pwb_h.at[1, pl.ds(c0, G // CHUNK)], pbl)
            prep(0, 0)

            def pair_body(k2, _):
                drain(0)
                wait_scatter(1)
                prep(2 * k2 + 1, 1)
                scale_scatter(0)
                drain(1)
                wait_scatter(0)

                @pl.when(k2 < npair - 1)
                def _():
                    prep(2 * k2 + 2, 0)

                scale_scatter(1)
                return 0

            lax.fori_loop(0, npair, pair_body, 0)
            return 0

        lax.fori_loop(0, ngrp, msg_group, 0)
        wait_scatter(1)
        plsc.subcore_barrier()

        for m in range(5):
            r0 = (s * 5 + m) * CHUNK
            pltpu.sync_copy(
                acc.at[pl.ds(r0, CHUNK)],
                out_h.at[c, pl.ds(pl.multiple_of(r0, 8), CHUNK)])

    return kb(uidx, iidx, pwb, emb_stack)


def _mm_relu_body(msg_ref, emb_ref, w_ref, out_ref):
    x = msg_ref[0] + emb_ref[0]
    y = lax.dot_general(
        x, w_ref[0],
        dimension_numbers=(((1,), (1,)), ((), ())),
        preferred_element_type=jnp.float32,
    )
    out_ref[0] = jnp.maximum(y, 0.0)


def _mm_relu(msg, emb, w):
    n = msg.shape[1]
    br = 2000
    return pl.pallas_call(
        _mm_relu_body,
        grid=(2, n // br),
        in_specs=[
            pl.BlockSpec((1, br, 128), lambda g, r: (g, r, 0)),
            pl.BlockSpec((1, br, 128), lambda g, r: (g, r, 0)),
            pl.BlockSpec((1, 128, 128), lambda g, r: (g, 0, 0)),
        ],
        out_specs=pl.BlockSpec((1, br, 128), lambda g, r: (g, r, 0)),
        out_shape=jax.ShapeDtypeStruct((2, n, 128), jnp.float32),
    )(msg, emb, w)


def kernel(u_emb, i_emb, edge_index, weights, W_u, W_i):
    e = edge_index.shape[1]
    per_round = G * NC * NS
    e_pad = -(-e // per_round) * per_round
    n_pad = e_pad - e

    uidx = edge_index[0]
    iidx = edge_index[1]
    if n_pad:
        pad_nodes = N_NODES + (
            jnp.arange(n_pad, dtype=jnp.int32) % (N_PAD_NODES - N_NODES))
        uidx = jnp.concatenate([uidx, pad_nodes])
        iidx = jnp.concatenate([iidx, pad_nodes])
        weights = jnp.concatenate([weights, jnp.zeros((n_pad,), jnp.float32)])

    pwb = _sc_factors(uidx, iidx, weights, e_pad)

    zpad = jnp.zeros((N_PAD_NODES - N_NODES, EMB), jnp.float32)
    emb_stack = jnp.concatenate([u_emb, zpad, i_emb, zpad])

    msg = _sc_messages(uidx, iidx, pwb, emb_stack, e_pad)
    msg = msg[:, :N_NODES]

    emb = jnp.stack([u_emb, i_emb])
    w = jnp.stack([W_u, W_i])
    out = _mm_relu(msg, emb, w)
    return (out[0], out[1])


# speedup vs baseline: 19.1983x; 1.0243x over previous
import functools

import jax
import jax.numpy as jnp
from jax import lax
from jax.experimental import pallas as pl
from jax.experimental.pallas import tpu as pltpu
from jax.experimental.pallas import tpu_sc as plsc

NC = 2
NS = 16
L = 16

N_NODES = 10000
N_PAD_NODES = 10240
EMB = 128
CHUNK = 128
G = 2048

_NO_LAYOUT = pltpu.CompilerParams(needs_layout_passes=False)


def _rsqrt_newton(d):
    xi = lax.bitcast_convert_type(d, jnp.int32)
    xi = 0x5F3759DF - lax.shift_right_logical(xi, 1)
    y = lax.bitcast_convert_type(xi, jnp.float32)
    for _ in range(3):
        y = y * (1.5 - 0.5 * d * y * y)
    return y


def _sc_factors(uidx, iidx, w, e_pad):
    ept16 = e_pad // NS
    mesh = plsc.VectorSubcoreMesh(core_axis_name="c", subcore_axis_name="s")

    @functools.partial(
        pl.kernel,
        out_type=jax.ShapeDtypeStruct((NC, e_pad // CHUNK, CHUNK), jnp.float32),
        mesh=mesh,
        compiler_params=_NO_LAYOUT,
        scratch_types=[
            pltpu.VMEM((ept16 // CHUNK, CHUNK), jnp.int32),
            pltpu.VMEM((ept16 // CHUNK, CHUNK), jnp.float32),
            pltpu.VMEM((ept16 // CHUNK, CHUNK), jnp.float32),
            pltpu.VMEM((80, EMB), jnp.float32),
            pltpu.VMEM((80,), jnp.int32),
            pltpu.VMEM((8, EMB), jnp.float32),
            pltpu.VMEM_SHARED((80, EMB), jnp.float32),
        ],
    )
    def ka(idx_h, w_h, out_h, gidx, wloc, fout, hist, rowids, tbl, deg):
        c = lax.axis_index("c")
        s = lax.axis_index("s")
        pos16 = lax.iota(jnp.int32, L)
        zeros16 = jnp.zeros((L,), jnp.float32)
        nck = ept16 // CHUNK
        c0 = pl.multiple_of((s * ept16) // CHUNK, 8)

        pltpu.sync_copy(idx_h.at[c, pl.ds(c0, nck)], gidx)
        pltpu.sync_copy(w_h.at[c, pl.ds(c0, nck)], wloc)

        def zero_body(r, _):
            for j in range(EMB // L):
                hist[r, pl.ds(j * L, L)] = zeros16
            return 0

        lax.fori_loop(0, 80, zero_body, 0)
        for r in range(5):
            rowids[pl.ds(r * L, L)] = pos16 + r * L
        for j in range(EMB // L):
            for r in range(5):
                tbl[r, pl.ds(j * L, L)] = zeros16
        pltpu.sync_copy(tbl.at[pl.ds(0, 5)], deg.at[pl.ds(s * 5, 5)])
        plsc.subcore_barrier()

        def chunk_body(kk, _):
            for g in range(CHUNK // L):
                v = gidx[kk, pl.ds(g * L, L)]
                cnt, last = plsc.scan_count(v)
                plsc.addupdate_scatter(
                    hist,
                    [lax.shift_right_logical(v, 7), v & (EMB - 1)],
                    cnt.astype(jnp.float32), mask=last)
            return 0

        lax.fori_loop(0, nck, chunk_body, 0)
        pltpu.sync_copy(hist, deg.at[rowids], add=True)
        plsc.subcore_barrier()

        pltpu.sync_copy(deg.at[pl.ds(s * 5, 5)], tbl.at[pl.ds(0, 5)])

        def rsq_body(r, _):
            for j in range(EMB // L):
                d = jnp.maximum(tbl[r, pl.ds(j * L, L)], 1.0)
                tbl[r, pl.ds(j * L, L)] = _rsqrt_newton(d)
            return 0

        lax.fori_loop(0, 5, rsq_body, 0)
        pltpu.sync_copy(tbl.at[pl.ds(0, 5)], deg.at[pl.ds(s * 5, 5)])
        plsc.subcore_barrier()

        pltpu.sync_copy(deg, hist)

        def fac_body(kk, _):
            for g in range(CHUNK // L):
                v = gidx[kk, pl.ds(g * L, L)]
                f = plsc.load_gather(
                    hist,
                    [lax.shift_right_logical(v, 7), v & (EMB - 1)])
                wv = wloc[kk, pl.ds(g * L, L)]
                fout[kk, pl.ds(g * L, L)] = f * wv
            return 0

        lax.fori_loop(0, nck, fac_body, 0)
        pltpu.sync_copy(fout, out_h.at[c, pl.ds(c0, nck)])

    idx_stack = jnp.stack([uidx, iidx]).reshape(NC, e_pad // CHUNK, CHUNK)
    ones = jnp.ones_like(w)
    w_stack = jnp.stack([w, ones]).reshape(NC, e_pad // CHUNK, CHUNK)
    return ka(idx_stack, w_stack)


def _sc_messages(gidx_all, didx_all, ew2d, emb_stack, e_pad):
    ept = e_pad // NS
    ngrp = ept // G
    npair = G // CHUNK // 2
    nck = G // CHUNK
    mesh = plsc.VectorSubcoreMesh(core_axis_name="c", subcore_axis_name="s")

    @functools.partial(
        pl.kernel,
        out_type=jax.ShapeDtypeStruct((NC, N_PAD_NODES, EMB), jnp.float32),
        mesh=mesh,
        compiler_params=_NO_LAYOUT,
        scratch_types=[
            pltpu.VMEM((G // CHUNK, CHUNK), jnp.int32),
            pltpu.VMEM((G // CHUNK, CHUNK), jnp.int32),
            pltpu.VMEM((G // CHUNK, CHUNK), jnp.float32),
            pltpu.VMEM((CHUNK, EMB), jnp.float32),
            pltpu.VMEM((CHUNK, EMB), jnp.float32),
            pltpu.VMEM_SHARED((N_PAD_NODES, EMB), jnp.float32),
            pltpu.SemaphoreType.DMA,
            pltpu.SemaphoreType.DMA,
            pltpu.SemaphoreType.DMA,
            pltpu.SemaphoreType.DMA,
        ],
    )
    def kb(gidx_h, didx_h, ew_h, emb_h, out_h,
           gl, dl, ewl, rows0, rows1, acc, sem0, sem1, ssem0, ssem1):
        c = lax.axis_index("c")
        s = lax.axis_index("s")
        zeros16 = jnp.zeros((L,), jnp.float32)
        bufs = ((rows0, sem0, ssem0), (rows1, sem1, ssem1))

        def start_gather(kk, b):
            rows, sem, ssem = bufs[b]
            pltpu.async_copy(emb_h.at[gl.at[kk]], rows, sem)

        def drain_gather(b):
            rows, sem, ssem = bufs[b]
            pltpu.make_async_copy(emb_h.at[gl.at[0]], rows, sem).wait()

        def start_scatter(kk, b):
            rows, sem, ssem = bufs[b]
            pltpu.async_copy(rows, acc.at[dl.at[kk]], ssem, add=True)

        def wait_scatter(b):
            rows, sem, ssem = bufs[b]
            pltpu.make_async_copy(rows, acc.at[dl.at[0]], ssem).wait()

        def scale(kk, b):
            rows, sem, ssem = bufs[b]

            def scale_body(g2, _):
                evec = ewl[kk, pl.ds(g2 * L, L)]
                for l in range(L):
                    sc = evec[l]
                    e = g2 * L + l
                    for j in range(EMB // L):
                        rows[e, pl.ds(j * L, L)] = (
                            rows[e, pl.ds(j * L, L)] * sc)
                return 0

            lax.fori_loop(0, CHUNK // L, scale_body, 0)

        def zero_body(r, _):
            for j in range(EMB // L):
                rows0[r, pl.ds(j * L, L)] = zeros16
            return 0

        lax.fori_loop(0, CHUNK, zero_body, 0)
        for m in range(5):
            pltpu.sync_copy(rows0, acc.at[pl.ds((s * 5 + m) * CHUNK, CHUNK)])
        plsc.subcore_barrier()

        def msg_group(g_id, _):
            off = s * ept + g_id * G
            c0 = pl.multiple_of(off // CHUNK, 8)
            pltpu.sync_copy(gidx_h.at[c, pl.ds(c0, nck)], gl)
            pltpu.sync_copy(didx_h.at[c, pl.ds(c0, nck)], dl)
            pltpu.sync_copy(ew_h.at[pl.ds(c0, nck)], ewl)
            start_gather(0, 0)

            def pair_body(k2, _):
                drain_gather(0)

                @pl.when(k2 > 0)
                def _():
                    wait_scatter(1)

                start_gather(2 * k2 + 1, 1)
                scale(2 * k2, 0)
                start_scatter(2 * k2, 0)
                drain_gather(1)
                wait_scatter(0)

                @pl.when(k2 < npair - 1)
                def _():
                    start_gather(2 * k2 + 2, 0)

                scale(2 * k2 + 1, 1)
                start_scatter(2 * k2 + 1, 1)
                return 0

            lax.fori_loop(0, npair, pair_body, 0)
            wait_scatter(1)
            return 0

        lax.fori_loop(0, ngrp, msg_group, 0)
        plsc.subcore_barrier()

        for m in range(5):
            r0 = (s * 5 + m) * CHUNK
            pltpu.sync_copy(
                acc.at[pl.ds(r0, CHUNK)],
                out_h.at[c, pl.ds(pl.multiple_of(r0, 8), CHUNK)])

    return kb(gidx_all, didx_all, ew2d, emb_stack)


def _mm_relu_body(msg_ref, emb_ref, w_ref, out_ref):
    x = msg_ref[0] + emb_ref[0]
    y = lax.dot_general(
        x, w_ref[0],
        dimension_numbers=(((1,), (1,)), ((), ())),
        preferred_element_type=jnp.float32,
    )
    out_ref[0] = jnp.maximum(y, 0.0)


def _mm_relu(msg, emb, w):
    n = msg.shape[1]
    br = 2000
    return pl.pallas_call(
        _mm_relu_body,
        grid=(2, n // br),
        in_specs=[
            pl.BlockSpec((1, br, 128), lambda g, r: (g, r, 0)),
            pl.BlockSpec((1, br, 128), lambda g, r: (g, r, 0)),
            pl.BlockSpec((1, 128, 128), lambda g, r: (g, 0, 0)),
        ],
        out_specs=pl.BlockSpec((1, br, 128), lambda g, r: (g, r, 0)),
        out_shape=jax.ShapeDtypeStruct((2, n, 128), jnp.float32),
    )(msg, emb, w)


def kernel(u_emb, i_emb, edge_index, weights, W_u, W_i):
    e = edge_index.shape[1]
    per_round = G * NC * NS
    e_pad = -(-e // per_round) * per_round
    n_pad = e_pad - e

    uidx = edge_index[0]
    iidx = edge_index[1]
    if n_pad:
        pad_nodes = N_NODES + (
            jnp.arange(n_pad, dtype=jnp.int32) % (N_PAD_NODES - N_NODES))
        uidx = jnp.concatenate([uidx, pad_nodes])
        iidx = jnp.concatenate([iidx, pad_nodes])
        weights = jnp.concatenate([weights, jnp.zeros((n_pad,), jnp.float32)])

    pwb = _sc_factors(uidx, iidx, weights, e_pad)
    ew2d = pwb[0] * pwb[1]

    shape3 = (e_pad // CHUNK, CHUNK)
    gidx_all = jnp.stack(
        [iidx.reshape(shape3) + N_PAD_NODES, uidx.reshape(shape3)])
    didx_all = jnp.stack([uidx.reshape(shape3), iidx.reshape(shape3)])

    zpad = jnp.zeros((N_PAD_NODES - N_NODES, EMB), jnp.float32)
    emb_stack = jnp.concatenate([u_emb, zpad, i_emb, zpad])

    msg = _sc_messages(gidx_all, didx_all, ew2d, emb_stack, e_pad)
    msg = msg[:, :N_NODES]

    emb = jnp.stack([u_emb, i_emb])
    w = jnp.stack([W_u, W_i])
    out = _mm_relu(msg, emb, w)
    return (out[0], out[1])
